# baseline (speedup 1.0000x reference)
import functools

import jax
import jax.numpy as jnp
from jax import lax
from jax.experimental import pallas as pl
from jax.experimental.pallas import tpu as pltpu
from jax.experimental.pallas import tpu_sc as plsc

_VOCAB = 1000000
_EMB = 64
_B = 4096
_L = 200
_N = _B * _L

_NW = 32
_CHUNK = 128
_NCHT = _N // _CHUNK
_NCH = _NCHT // _NW
_BB = _B // _CHUNK
_NSB = 4
_LEAD = 2
_NOB = 2


def _make_emb_kernel():
    mesh = plsc.VectorSubcoreMesh(core_axis_name="c", subcore_axis_name="s")

    @functools.partial(
        pl.kernel,
        mesh=mesh,
        out_type=jax.ShapeDtypeStruct((_L, 8, _BB, 8, _CHUNK), jnp.float32),
        compiler_params=pltpu.CompilerParams(
            use_tc_tiling_on_sc=False, needs_layout_passes=False
        ),
        scratch_types=[
            pltpu.VMEM((_NCH, _CHUNK), jnp.int32),
            pltpu.VMEM((_NSB, _CHUNK), jnp.int32),
            *[pltpu.VMEM((_CHUNK, 128), jnp.float32) for _ in range(_NSB)],
            *[pltpu.VMEM((8, 8, _CHUNK), jnp.float32) for _ in range(_NOB)],
            *[pltpu.SemaphoreType.DMA for _ in range(_NSB + _NOB)],
        ],
    )
    def emb(xt_hbm, table2_hbm, out_hbm, idx_v, sring, *rest):
        sbufs = rest[:_NSB]
        obufs = rest[_NSB:_NSB + _NOB]
        gsems = rest[_NSB + _NOB:2 * _NSB + _NOB]
        osems = rest[2 * _NSB + _NOB:]
        wid = lax.axis_index("s") * 2 + lax.axis_index("c")
        m0 = wid * _NCH

        pltpu.sync_copy(xt_hbm.at[pl.ds(m0, _NCH)], idx_v)

        riotas = [jax.lax.iota(jnp.int32, 16) + 16 * t for t in range(8)]

        def compute_sidx(j, slot):
            for t in range(8):
                v = idx_v[j, pl.ds(16 * t, 16)]
                sring[slot, pl.ds(16 * t, 16)] = jax.lax.shift_right_logical(v, 1)

        def start_gather(j, slot):
            compute_sidx(j, slot)
            return pltpu.async_copy(
                table2_hbm.at[sring.at[slot]], sbufs[slot], gsems[slot]
            )

        for j in range(_LEAD):
            start_gather(j, j % _NSB)

        @pl.loop(0, _NCH, step=_NSB)
        def group(g):
            for bslot in range(_NSB):
                j = g + bslot
                o = bslot % _NOB
                pltpu.make_async_copy(
                    table2_hbm.at[sring.at[bslot]], sbufs[bslot], gsems[bslot]
                ).wait()

                nslot = (bslot + _LEAD) % _NSB

                @pl.when(j + _LEAD < _NCH)
                def _():
                    start_gather(j + _LEAD, nslot)

                @pl.when(j >= _NOB)
                def _():
                    pltpu.make_async_copy(
                        obufs[o], out_hbm.at[0, :, 0], osems[o]
                    ).wait()

                colbases = []
                for t in range(8):
                    pv = jnp.bitwise_and(idx_v[j, pl.ds(16 * t, 16)], 1)
                    colbases.append(pv * 64)

                for cb in range(8):
                    for c8 in range(8):
                        for t in range(8):
                            cols = colbases[t] + (cb * 8 + c8)
                            vec = plsc.load_gather(
                                sbufs[bslot], [riotas[t], cols]
                            )
                            obufs[o][cb, c8, pl.ds(16 * t, 16)] = vec

                m = m0 + j
                l = m // _BB
                bb = m % _BB
                pltpu.async_copy(obufs[o], out_hbm.at[l, :, bb], osems[o])

        for o in range(_NOB):
            pltpu.make_async_copy(
                obufs[o], out_hbm.at[0, :, 0], osems[o]
            ).wait()

    return emb


_emb = _make_emb_kernel()


def kernel(x, weight):
    xt = jnp.transpose(x).reshape(_NCHT, _CHUNK).astype(jnp.int32)
    table2 = weight.reshape(_VOCAB // 2, 128)
    out5 = _emb(xt, table2)
    return out5.transpose(2, 4, 0, 1, 3).reshape(_B, _L, _EMB)


# speedup vs baseline: 1.4881x; 1.4881x over previous
import functools

import jax
import jax.numpy as jnp
from jax import lax
from jax.experimental import pallas as pl
from jax.experimental.pallas import tpu as pltpu
from jax.experimental.pallas import tpu_sc as plsc

_VOCAB = 1000000
_EMB = 64
_B = 4096
_L = 200
_N = _B * _L

_NW = 32
_CHUNK = 128
_NCHT = _N // _CHUNK
_NCH = _NCHT // _NW
_BB = _B // _CHUNK
_NSB = 4
_LEAD = 2
_NOB = 2


def _make_emb_kernel():
    mesh = plsc.VectorSubcoreMesh(core_axis_name="c", subcore_axis_name="s")

    @functools.partial(
        pl.kernel,
        mesh=mesh,
        out_type=jax.ShapeDtypeStruct((_L, 8, _BB, 8, _CHUNK), jnp.float32),
        compiler_params=pltpu.CompilerParams(
            use_tc_tiling_on_sc=False, needs_layout_passes=False
        ),
        scratch_types=[
            pltpu.VMEM((_NCH, _CHUNK), jnp.int32),
            pltpu.VMEM((_NSB, _CHUNK), jnp.int32),
            *[pltpu.VMEM((_CHUNK, 128), jnp.float32) for _ in range(_NSB)],
            *[pltpu.VMEM((8, 8, _CHUNK), jnp.float32) for _ in range(_NOB)],
            *[pltpu.SemaphoreType.DMA for _ in range(_NSB + _NOB)],
        ],
    )
    def emb(xt_hbm, table2_hbm, out_hbm, idx_v, sring, *rest):
        sbufs = rest[:_NSB]
        obufs = rest[_NSB:_NSB + _NOB]
        gsems = rest[_NSB + _NOB:2 * _NSB + _NOB]
        osems = rest[2 * _NSB + _NOB:]
        wid = lax.axis_index("s") * 2 + lax.axis_index("c")
        m0 = wid * _NCH

        pltpu.sync_copy(xt_hbm.at[pl.ds(m0, _NCH)], idx_v)

        riotas = [jax.lax.iota(jnp.int32, 16) + 16 * t for t in range(8)]

        def compute_sidx(j, slot):
            for t in range(8):
                v = idx_v[j, pl.ds(16 * t, 16)]
                sring[slot, pl.ds(16 * t, 16)] = jax.lax.shift_right_logical(v, 1)

        def start_gather(j, slot):
            compute_sidx(j, slot)
            return pltpu.async_copy(
                table2_hbm.at[sring.at[slot]], sbufs[slot], gsems[slot]
            )

        for j in range(_LEAD):
            start_gather(j, j % _NSB)

        @pl.loop(0, _NCH, step=_NSB)
        def group(g):
            for bslot in range(_NSB):
                j = g + bslot
                o = bslot % _NOB
                pltpu.make_async_copy(
                    table2_hbm.at[sring.at[bslot]], sbufs[bslot], gsems[bslot]
                ).wait()

                nslot = (bslot + _LEAD) % _NSB

                @pl.when(j + _LEAD < _NCH)
                def _():
                    start_gather(j + _LEAD, nslot)

                @pl.when(j >= _NOB)
                def _():
                    pltpu.make_async_copy(
                        obufs[o], out_hbm.at[0, :, 0], osems[o]
                    ).wait()

                colbases = []
                for t in range(8):
                    pv = jnp.bitwise_and(idx_v[j, pl.ds(16 * t, 16)], 1)
                    colbases.append(pv * 64)

                @plsc.parallel_loop(0, 64, unroll=8)
                def _(c):
                    cb = c // 8
                    c8 = c % 8
                    for t in range(8):
                        cols = colbases[t] + c
                        vec = plsc.load_gather(
                            sbufs[bslot], [riotas[t], cols]
                        )
                        obufs[o][cb, c8, pl.ds(16 * t, 16)] = vec

                m = m0 + j
                l = m // _BB
                bb = m % _BB
                pltpu.async_copy(obufs[o], out_hbm.at[l, :, bb], osems[o])

        for o in range(_NOB):
            pltpu.make_async_copy(
                obufs[o], out_hbm.at[0, :, 0], osems[o]
            ).wait()

    return emb


_emb = _make_emb_kernel()


def kernel(x, weight):
    xt = jnp.transpose(x).reshape(_NCHT, _CHUNK).astype(jnp.int32)
    table2 = weight.reshape(_VOCAB // 2, 128)
    out5 = _emb(xt, table2)
    return out5.transpose(2, 4, 0, 1, 3).reshape(_B, _L, _EMB)


# speedup vs baseline: 1.6117x; 1.0831x over previous
import functools

import jax
import jax.numpy as jnp
from jax import lax
from jax.experimental import pallas as pl
from jax.experimental.pallas import tpu as pltpu
from jax.experimental.pallas import tpu_sc as plsc

_VOCAB = 1000000
_EMB = 64
_B = 4096
_L = 200
_N = _B * _L

_NW = 32
_PER_W = _N // _NW
_CHUNK = 128
_NCH = _PER_W // _CHUNK
_NBUF = 8
_LEAD = 4


def _make_emb_kernel():
    mesh = plsc.VectorSubcoreMesh(core_axis_name="c", subcore_axis_name="s")

    @functools.partial(
        pl.kernel,
        mesh=mesh,
        out_type=jax.ShapeDtypeStruct((_N, _EMB), jnp.float32),
        compiler_params=pltpu.CompilerParams(use_tc_tiling_on_sc=False),
        scratch_types=[
            pltpu.VMEM((_NCH, _CHUNK), jnp.int32),
            *[pltpu.VMEM((_CHUNK, _EMB), jnp.float32) for _ in range(_NBUF)],
            *[pltpu.SemaphoreType.DMA for _ in range(2 * _NBUF)],
        ],
    )
    def emb(idx_hbm, table_hbm, out_hbm, idx_v, *rest):
        bufs = rest[:_NBUF]
        gsems = rest[_NBUF:2 * _NBUF]
        wsems = rest[2 * _NBUF:]
        wid = lax.axis_index("s") * 2 + lax.axis_index("c")
        base = wid * _PER_W
        pltpu.sync_copy(idx_hbm.at[pl.ds(wid * _NCH, _NCH)], idx_v)

        for b in range(_LEAD):
            pltpu.async_copy(table_hbm.at[idx_v.at[b]], bufs[b], gsems[b])

        @pl.loop(0, _NCH, step=_NBUF)
        def group(g):
            for b in range(_NBUF):
                j = g + b
                nb = (b + _LEAD) % _NBUF
                pltpu.make_async_copy(
                    table_hbm.at[idx_v.at[0]], bufs[b], gsems[b]
                ).wait()
                pltpu.async_copy(
                    bufs[b], out_hbm.at[pl.ds(base + j * _CHUNK, _CHUNK)],
                    wsems[b],
                )
                @pl.when(jnp.logical_and(j >= _LEAD, j + _LEAD < _NCH))
                def _():
                    pltpu.make_async_copy(
                        bufs[nb], out_hbm.at[pl.ds(0, _CHUNK)], wsems[nb]
                    ).wait()

                @pl.when(j + _LEAD < _NCH)
                def _():
                    pltpu.async_copy(
                        table_hbm.at[idx_v.at[j + _LEAD]], bufs[nb], gsems[nb]
                    )

        for b in range(_NBUF):
            pltpu.make_async_copy(
                bufs[b], out_hbm.at[pl.ds(0, _CHUNK)], wsems[b]
            ).wait()

    return emb


_emb = _make_emb_kernel()


def kernel(x, weight):
    idx = x.reshape(_N // _CHUNK, _CHUNK).astype(jnp.int32)
    out = _emb(idx, weight)
    return out.reshape(_B, _L, _EMB)
